# baseline (speedup 1.0000x reference)
import functools

import jax
import jax.numpy as jnp
from jax import lax
from jax.experimental import pallas as pl
from jax.experimental.pallas import tpu as pltpu
from jax.experimental.pallas import tpu_sc as plsc

B = 16384
N_DENSE = 13
N_SPARSE = 26
VOCAB = 100000
EMB = 32
N_CROSS = 3
OUT_DIM = 64
X_DIM = N_DENSE + N_SPARSE * EMB

_SC_CORES = 2
_SC_SUBCORES = 16
_NW = _SC_CORES * _SC_SUBCORES

_N_ROWS = B * N_SPARSE
_PER_W = _N_ROWS // _NW
_IDXW = 128
_CHUNK = 256
_SLICES = _CHUNK // _IDXW
_HALVES = 8 // _SLICES
_SUPER = _HALVES * _CHUNK
_N_SUPER = _PER_W // _SUPER
_L = 16


def _sc_gather(table4, flat_idx2d):
    mesh = plsc.VectorSubcoreMesh(core_axis_name="c", subcore_axis_name="s")

    @functools.partial(
        pl.kernel,
        mesh=mesh,
        compiler_params=pltpu.CompilerParams(use_tc_tiling_on_sc=True,
                                             needs_layout_passes=False),
        out_type=jax.ShapeDtypeStruct((_N_ROWS, EMB), jnp.float32),
        scratch_types=[
            pltpu.VMEM((8, _IDXW), jnp.int32),
            pltpu.VMEM((8, _IDXW), jnp.int32),
            pltpu.VMEM((_CHUNK, 128), jnp.float32),
            pltpu.VMEM((_CHUNK, EMB), jnp.float32),
            pltpu.SemaphoreType.DMA,
        ],
    )
    def gather_k(table_hbm, idx_hbm, out_hbm, idx_v, idx4_v, rows4_v,
                 rows_v, sem):
        wid = lax.axis_index("s") * _SC_CORES + lax.axis_index("c")
        base = wid * _PER_W

        def super_body(i, carry):
            soff = base + i * _SUPER
            row0 = pl.multiple_of(soff // _IDXW, 8)
            pltpu.sync_copy(idx_hbm.at[pl.ds(row0, 8)], idx_v)
            for r in range(8):
                for c in range(_IDXW // _L):
                    idx4_v[r, pl.ds(c * _L, _L)] = (
                        idx_v[r, pl.ds(c * _L, _L)] >> 2)
            for half in range(_HALVES):
                off = soff + half * _CHUNK
                for j in range(_SLICES):
                    pltpu.async_copy(
                        table_hbm.at[idx4_v.at[half * _SLICES + j]],
                        rows4_v.at[pl.ds(j * _IDXW, _IDXW)], sem)
                for j in range(_SLICES):
                    pltpu.make_async_copy(
                        table_hbm.at[idx4_v.at[half * _SLICES + j]],
                        rows4_v.at[pl.ds(j * _IDXW, _IDXW)], sem).wait()

                def ext(g, c):
                    r = half * _SLICES + g // (_IDXW // _L)
                    col = (g % (_IDXW // _L)) * _L
                    sub = (idx_v[r, pl.ds(col, _L)] & 3) * EMB
                    rows16 = lax.iota(jnp.int32, _L) + g * _L
                    for w in range(EMB):
                        vals = plsc.load_gather(
                            rows4_v, [rows16, sub + w])
                        plsc.store_scatter(
                            rows_v, [rows16,
                                     jnp.zeros((_L,), jnp.int32) + w], vals)
                    return c
                lax.fori_loop(0, _CHUNK // _L, ext, 0)
                pltpu.sync_copy(rows_v, out_hbm.at[pl.ds(off, _CHUNK)])
            return carry

        lax.fori_loop(0, _N_SUPER, super_body, 0)

    return gather_k(table4, flat_idx2d)


_S = 1000
_SUPER_V = 4 * _S
_NSUP = VOCAB // _SUPER_V
_VQ = VOCAB // 4


def _repack_block(t_ref, out_ref):
    x = t_ref[0]
    out_ref[...] = jnp.concatenate(
        [x[q * _S:(q + 1) * _S, :] for q in range(4)], axis=1)


def _tc_repack(embed_tables):
    return pl.pallas_call(
        _repack_block,
        grid=(N_SPARSE, _NSUP),
        in_specs=[pl.BlockSpec((1, _SUPER_V, EMB), lambda f, s: (f, s, 0))],
        out_specs=pl.BlockSpec((_S, 128), lambda f, s: (f * _NSUP + s, 0)),
        out_shape=jax.ShapeDtypeStruct((N_SPARSE * VOCAB // 4, 128),
                                       jnp.float32),
    )(embed_tables)


_BT = 1024


def _dcn_block(inp_ref, emb_ref, cw_ref, b1_ref, w1_ref, w2_ref, b2_ref,
               w3_ref, b3_ref, wo_ref, sc_ref, out_ref):
    x = jnp.concatenate([inp_ref[:, :N_DENSE], emb_ref[...]], axis=1)

    h = jnp.maximum(
        jnp.dot(x, w1_ref[...], preferred_element_type=jnp.float32)
        + b1_ref[...], 0.0)
    h = jnp.maximum(
        jnp.dot(h, w2_ref[...], preferred_element_type=jnp.float32)
        + b2_ref[...], 0.0)
    dnn = jnp.maximum(
        jnp.dot(h, w3_ref[...], preferred_element_type=jnp.float32)
        + b3_ref[...], 0.0)

    xl = x
    for i in range(N_CROSS):
        alpha = jnp.dot(xl, cw_ref[:, i:i + 1],
                        preferred_element_type=jnp.float32)
        xl = (x * alpha + sc_ref[:, i:i + 1]) + xl

    cat = jnp.concatenate([xl, dnn], axis=1)
    logit = jnp.dot(cat, wo_ref[...],
                    preferred_element_type=jnp.float32) + sc_ref[:, 3:4]
    out_ref[...] = jax.nn.sigmoid(logit)


def _tc_dcn(inputs, emb, cw, b1, w1, w2, b2, w3, b3, wo, sc,
            interpret=False):
    grid = (B // _BT,)

    def full(shape):
        return pl.BlockSpec(shape, lambda i: tuple(0 for _ in shape))

    return pl.pallas_call(
        _dcn_block,
        grid=grid,
        in_specs=[
            pl.BlockSpec((_BT, N_DENSE + N_SPARSE), lambda i: (i, 0)),
            pl.BlockSpec((_BT, N_SPARSE * EMB), lambda i: (i, 0)),
            full(cw.shape),
            full(b1.shape),
            full(w1.shape),
            full(w2.shape),
            full(b2.shape),
            full(w3.shape),
            full(b3.shape),
            full(wo.shape),
            full(sc.shape),
        ],
        out_specs=pl.BlockSpec((_BT, 1), lambda i: (i, 0)),
        out_shape=jax.ShapeDtypeStruct((B, 1), jnp.float32),
        interpret=interpret,
    )(inputs, emb, cw, b1, w1, w2, b2, w3, b3, wo, sc)


def kernel(inputs, embed_tables, cross_w, cross_b, W1, b1, W2, b2, W3, b3,
           Wo, bo):
    v = inputs[:, N_DENSE:].astype(jnp.int32)
    f = jnp.arange(N_SPARSE, dtype=jnp.int32)[None, :]
    row = f * _VQ + (v // _SUPER_V) * _S + v % _S
    sub = (v // _S) % 4
    flat_idx = (row * 4 + sub).reshape(_N_ROWS // _IDXW, _IDXW)
    table4 = _tc_repack(embed_tables)

    emb = _sc_gather(table4, flat_idx).reshape(B, N_SPARSE * EMB)

    cw = jnp.concatenate([cross_w[0], cross_w[1], cross_w[2]], axis=1)
    sc = jnp.concatenate([cross_b.reshape(-1), bo.reshape(-1)]).reshape(1, 4)
    return _tc_dcn(inputs, emb, cw, b1.reshape(1, -1), W1,
                   W2, b2.reshape(1, -1), W3, b3.reshape(1, -1), Wo, sc)


# speedup vs baseline: 1.7902x; 1.7902x over previous
import functools

import jax
import jax.numpy as jnp
from jax import lax
from jax.experimental import pallas as pl
from jax.experimental.pallas import tpu as pltpu
from jax.experimental.pallas import tpu_sc as plsc

B = 16384
N_DENSE = 13
N_SPARSE = 26
VOCAB = 100000
EMB = 32
N_CROSS = 3
OUT_DIM = 64
X_DIM = N_DENSE + N_SPARSE * EMB

_SC_CORES = 2
_SC_SUBCORES = 16
_NW = _SC_CORES * _SC_SUBCORES

_N_ROWS = B * N_SPARSE
_PER_W = _N_ROWS // _NW
_IDXW = 128
_SLICES = 8
_CHUNK = _SLICES * _IDXW
_N_CHUNKS = _PER_W // _CHUNK


def _sc_gather(table_flat, flat_idx2d):
    mesh = plsc.VectorSubcoreMesh(core_axis_name="c", subcore_axis_name="s")

    @functools.partial(
        pl.kernel,
        mesh=mesh,
        compiler_params=pltpu.CompilerParams(use_tc_tiling_on_sc=False),
        out_type=jax.ShapeDtypeStruct((_N_ROWS, EMB), jnp.float32),
        scratch_types=[
            pltpu.VMEM((_SLICES, _IDXW), jnp.int32),
            pltpu.VMEM((_CHUNK, EMB), jnp.float32),
            pltpu.SemaphoreType.DMA,
        ],
    )
    def gather_k(table_hbm, idx_hbm, out_hbm, idx_v, rows_v, sem):
        wid = lax.axis_index("s") * _SC_CORES + lax.axis_index("c")
        base = wid * _PER_W

        def chunk_body(i, carry):
            off = base + i * _CHUNK
            pltpu.sync_copy(idx_hbm.at[pl.ds(off // _IDXW, _SLICES)], idx_v)
            for j in range(_SLICES):
                pltpu.async_copy(
                    table_hbm.at[idx_v.at[j]],
                    rows_v.at[pl.ds(j * _IDXW, _IDXW)], sem)
            for j in range(_SLICES):
                pltpu.make_async_copy(
                    table_hbm.at[idx_v.at[j]],
                    rows_v.at[pl.ds(j * _IDXW, _IDXW)], sem).wait()
            pltpu.sync_copy(rows_v, out_hbm.at[pl.ds(off, _CHUNK)])
            return carry

        lax.fori_loop(0, _N_CHUNKS, chunk_body, 0)

    return gather_k(table_flat, flat_idx2d)


_BT = 1024


def _dcn_block(inp_ref, emb_ref, cw_ref, b1_ref, w1_ref, w2_ref, b2_ref,
               w3_ref, b3_ref, wo_ref, sc_ref, out_ref):
    x = jnp.concatenate([inp_ref[:, :N_DENSE], emb_ref[...]], axis=1)

    h = jnp.maximum(
        jnp.dot(x, w1_ref[...], preferred_element_type=jnp.float32)
        + b1_ref[...], 0.0)
    h = jnp.maximum(
        jnp.dot(h, w2_ref[...], preferred_element_type=jnp.float32)
        + b2_ref[...], 0.0)
    dnn = jnp.maximum(
        jnp.dot(h, w3_ref[...], preferred_element_type=jnp.float32)
        + b3_ref[...], 0.0)

    xl = x
    for i in range(N_CROSS):
        alpha = jnp.dot(xl, cw_ref[:, i:i + 1],
                        preferred_element_type=jnp.float32)
        xl = (x * alpha + sc_ref[:, i:i + 1]) + xl

    cat = jnp.concatenate([xl, dnn], axis=1)
    logit = jnp.dot(cat, wo_ref[...],
                    preferred_element_type=jnp.float32) + sc_ref[:, 3:4]
    out_ref[...] = jax.nn.sigmoid(logit)


def _tc_dcn(inputs, emb, cw, b1, w1, w2, b2, w3, b3, wo, sc,
            interpret=False):
    grid = (B // _BT,)

    def full(shape):
        return pl.BlockSpec(shape, lambda i: tuple(0 for _ in shape))

    return pl.pallas_call(
        _dcn_block,
        grid=grid,
        in_specs=[
            pl.BlockSpec((_BT, N_DENSE + N_SPARSE), lambda i: (i, 0)),
            pl.BlockSpec((_BT, N_SPARSE * EMB), lambda i: (i, 0)),
            full(cw.shape),
            full(b1.shape),
            full(w1.shape),
            full(w2.shape),
            full(b2.shape),
            full(w3.shape),
            full(b3.shape),
            full(wo.shape),
            full(sc.shape),
        ],
        out_specs=pl.BlockSpec((_BT, 1), lambda i: (i, 0)),
        out_shape=jax.ShapeDtypeStruct((B, 1), jnp.float32),
        interpret=interpret,
    )(inputs, emb, cw, b1, w1, w2, b2, w3, b3, wo, sc)


def kernel(inputs, embed_tables, cross_w, cross_b, W1, b1, W2, b2, W3, b3,
           Wo, bo):
    idx = inputs[:, N_DENSE:].astype(jnp.int32)
    flat_idx = (idx + jnp.arange(N_SPARSE, dtype=jnp.int32)[None, :]
                * VOCAB).reshape(_N_ROWS // _IDXW, _IDXW)
    table_flat = embed_tables.reshape(N_SPARSE * VOCAB, EMB)

    emb = _sc_gather(table_flat, flat_idx).reshape(B, N_SPARSE * EMB)

    cw = jnp.concatenate([cross_w[0], cross_w[1], cross_w[2]], axis=1)
    sc = jnp.concatenate([cross_b.reshape(-1), bo.reshape(-1)]).reshape(1, 4)
    return _tc_dcn(inputs, emb, cw, b1.reshape(1, -1), W1,
                   W2, b2.reshape(1, -1), W3, b3.reshape(1, -1), Wo, sc)
